# baseline (speedup 1.0000x reference)
import functools

import jax
import jax.numpy as jnp
from jax import lax
from jax.experimental import pallas as pl
from jax.experimental.pallas import tpu as pltpu
from jax.experimental.pallas import tpu_sc as plsc

N = 100000
E = 1600000
D_NODE = 128
D_EDGE = 16
D_GLOB = 64
B = 256

NC = 2
NS = 16
NW = NC * NS
L = 16

UE = 800
EU = E // UE
K_E = (EU + NW - 1) // NW
UN = 160
NU = N // UN
K_N = (NU + NW - 1) // NW


def _sc_body(src_hbm, attr_hbm, x_hbm, batch_hbm, ze_hbm, zn_hbm, zf_hbm,
             esum_hbm, nsum_hbm, ecnt_hbm, ncnt_hbm,
             batch_spm, eacc_spm, nacc_spm,
             idx0, idx1, seg0, seg1, at0, at1, ar0, ar1, x0, x1, nb0, nb1,
             hist, cnt_v, si0, si1, sa0, sa1, sg0, sg1, ss0, ss1):
    c = lax.axis_index("c")
    s = lax.axis_index("s")
    wid = s * NC + c

    @pl.when(s == 0)
    def _():
        pltpu.sync_copy(batch_hbm, batch_spm)

    pltpu.sync_copy(ze_hbm, eacc_spm.at[pl.ds(s * (B // NS), B // NS)])
    pltpu.sync_copy(zn_hbm, nacc_spm.at[pl.ds(s * (B // NS), B // NS)])
    pltpu.sync_copy(zf_hbm, hist)
    plsc.subcore_barrier()

    lane = lax.broadcasted_iota(jnp.int32, (L,), 0) * B
    rows16 = lax.broadcasted_iota(jnp.int32, (L,), 0)
    ones = jnp.ones((L,), jnp.int32)

    def histo(segb, n):
        for i in range(n // L):
            segs = segb[pl.ds(i * L, L)]
            plsc.addupdate_scatter(hist, [lane + segs], ones)

    def cnt_out(out):
        for b in range(B // L):
            acc = hist[pl.ds(b * L, L)]
            for l in range(1, L):
                acc = acc + hist[pl.ds(l * B + b * L, L)]
            cnt_v[pl.ds(b * L, L)] = acc
        pltpu.sync_copy(cnt_v, out.at[wid])

    def transpose(atb, arb):
        def gbody(g, carry):
            off = pl.multiple_of(g * L, L)
            rows = rows16 + g * L
            for f in range(D_EDGE):
                vals = atb[f, pl.ds(off, L)]
                plsc.store_scatter(arb, [rows, jnp.full((L,), f, jnp.int32)],
                                   vals)
            return carry

        lax.fori_loop(0, UE // L, gbody, 0)

    def e_loads(t, idxb, atb, s_i, s_a):
        base = pl.multiple_of(t * UE, UE)
        di = pltpu.async_copy(src_hbm.at[pl.ds(base, UE)], idxb, s_i)
        da = pltpu.async_copy(attr_hbm.at[:, pl.ds(base, UE)], atb, s_a)
        return di, da

    def edge_pair(j, carry):
        t0 = (2 * j) * NW + wid
        t1 = t0 + NW
        di0, da0 = e_loads(t0, idx0, at0, si0, sa0)
        di0.wait()
        dg0 = pltpu.async_copy(batch_spm.at[idx0], seg0, sg0)
        di1, da1 = e_loads(t1, idx1, at1, si1, sa1)
        da0.wait()
        transpose(at0, ar0)
        dg0.wait()
        ds0 = pltpu.async_copy(ar0, eacc_spm.at[seg0], ss0, add=True)
        histo(seg0, UE)
        di1.wait()
        dg1 = pltpu.async_copy(batch_spm.at[idx1], seg1, sg1)
        da1.wait()
        transpose(at1, ar1)
        dg1.wait()
        ds1 = pltpu.async_copy(ar1, eacc_spm.at[seg1], ss1, add=True)
        histo(seg1, UE)
        ds0.wait()
        ds1.wait()
        return carry

    lax.fori_loop(0, (K_E - 1) // 2, edge_pair, 0)

    t_tail = (K_E - 1) * NW + wid

    @pl.when(t_tail < EU)
    def _():
        di0, da0 = e_loads(t_tail, idx0, at0, si0, sa0)
        di0.wait()
        pltpu.async_copy(batch_spm.at[idx0], seg0, sg0).wait()
        da0.wait()
        transpose(at0, ar0)
        ds0 = pltpu.async_copy(ar0, eacc_spm.at[seg0], ss0, add=True)
        histo(seg0, UE)
        ds0.wait()

    cnt_out(ecnt_hbm)
    pltpu.sync_copy(zf_hbm, hist)

    def n_loads(t, nbb, xb, s_i, s_a):
        base = pl.multiple_of(t * UN, UN)
        dn = pltpu.async_copy(batch_hbm.at[pl.ds(base, UN)], nbb, s_i)
        dx = pltpu.async_copy(x_hbm.at[pl.ds(base, UN)], xb, s_a)
        return dn, dx

    def node_pair(j, carry):
        t0 = (2 * j) * NW + wid
        t1 = t0 + NW
        dn0, dx0 = n_loads(t0, nb0, x0, si0, sa0)

        @pl.when(t1 < NU)
        def _():
            n_loads(t1, nb1, x1, si1, sa1)

        dn0.wait()
        dx0.wait()
        ds0 = pltpu.async_copy(x0, nacc_spm.at[nb0], ss0, add=True)
        histo(nb0, UN)

        @pl.when(t1 < NU)
        def _():
            base = pl.multiple_of(t1 * UN, UN)
            pltpu.make_async_copy(batch_hbm.at[pl.ds(base, UN)], nb1, si1).wait()
            pltpu.make_async_copy(x_hbm.at[pl.ds(base, UN)], x1, sa1).wait()
            ds1 = pltpu.async_copy(x1, nacc_spm.at[nb1], ss1, add=True)
            histo(nb1, UN)
            ds1.wait()

        ds0.wait()
        return carry

    lax.fori_loop(0, K_N // 2, node_pair, 0)

    cnt_out(ncnt_hbm)

    plsc.subcore_barrier()

    @pl.when(s == 0)
    def _():
        pltpu.sync_copy(eacc_spm, esum_hbm.at[c])
        pltpu.sync_copy(nacc_spm, nsum_hbm.at[c])


def _sc_segment_sums(edge_src, edge_attr_t, x, batch):
    mesh = plsc.VectorSubcoreMesh(core_axis_name="c", subcore_axis_name="s",
                                  num_cores=NC, num_subcores=NS)
    ze = jnp.zeros((B // NS, D_EDGE), jnp.float32)
    zn = jnp.zeros((B // NS, D_NODE), jnp.float32)
    zf = jnp.zeros((L * B,), jnp.int32)
    out_type = (
        jax.ShapeDtypeStruct((NC, B, D_EDGE), jnp.float32),
        jax.ShapeDtypeStruct((NC, B, D_NODE), jnp.float32),
        jax.ShapeDtypeStruct((NW, B), jnp.int32),
        jax.ShapeDtypeStruct((NW, B), jnp.int32),
    )
    scratch = [
        pltpu.VMEM_SHARED((N,), jnp.int32),
        pltpu.VMEM_SHARED((B, D_EDGE), jnp.float32),
        pltpu.VMEM_SHARED((B, D_NODE), jnp.float32),
        pltpu.VMEM((UE,), jnp.int32),
        pltpu.VMEM((UE,), jnp.int32),
        pltpu.VMEM((UE,), jnp.int32),
        pltpu.VMEM((UE,), jnp.int32),
        pltpu.VMEM((D_EDGE, UE), jnp.float32),
        pltpu.VMEM((D_EDGE, UE), jnp.float32),
        pltpu.VMEM((UE, D_EDGE), jnp.float32),
        pltpu.VMEM((UE, D_EDGE), jnp.float32),
        pltpu.VMEM((UN, D_NODE), jnp.float32),
        pltpu.VMEM((UN, D_NODE), jnp.float32),
        pltpu.VMEM((UN,), jnp.int32),
        pltpu.VMEM((UN,), jnp.int32),
        pltpu.VMEM((L * B,), jnp.int32),
        pltpu.VMEM((B,), jnp.int32),
        pltpu.SemaphoreType.DMA,
        pltpu.SemaphoreType.DMA,
        pltpu.SemaphoreType.DMA,
        pltpu.SemaphoreType.DMA,
        pltpu.SemaphoreType.DMA,
        pltpu.SemaphoreType.DMA,
        pltpu.SemaphoreType.DMA,
        pltpu.SemaphoreType.DMA,
    ]
    fn = pl.kernel(_sc_body, out_type=out_type, mesh=mesh,
                   scratch_types=scratch,
                   compiler_params=pltpu.CompilerParams(
                       needs_layout_passes=False,
                       use_tc_tiling_on_sc=False))
    return fn(edge_src, edge_attr_t, x, batch, ze, zn, zf)


def _tc_body(u_ref, esum_ref, nsum_ref, ecnt_ref, ncnt_ref,
             w1u_ref, w1e_ref, w1n_ref, b1_ref, w2_ref, b2_ref,
             gamma_ref, beta_ref, out_ref):
    e_sum = esum_ref[0] + esum_ref[1]
    n_sum = nsum_ref[0] + nsum_ref[1]
    e_cnt = jnp.sum(ecnt_ref[...], axis=0).astype(jnp.float32)
    n_cnt = jnp.sum(ncnt_ref[...], axis=0).astype(jnp.float32)
    e_mean = e_sum / jnp.maximum(e_cnt, 1.0)[:, None]
    n_mean = n_sum / jnp.maximum(n_cnt, 1.0)[:, None]
    u = u_ref[...]
    h = (jnp.dot(u, w1u_ref[...], preferred_element_type=jnp.float32)
         + jnp.dot(e_mean, w1e_ref[...], preferred_element_type=jnp.float32)
         + jnp.dot(n_mean, w1n_ref[...], preferred_element_type=jnp.float32)
         + b1_ref[...])
    h = jnp.maximum(h, 0.0)
    y = jnp.dot(h, w2_ref[...], preferred_element_type=jnp.float32) + b2_ref[...] + u
    mu = jnp.mean(y, axis=-1, keepdims=True)
    var = jnp.mean((y - mu) ** 2, axis=-1, keepdims=True)
    y = (y - mu) * lax.rsqrt(var + 1e-5)
    out_ref[...] = y * gamma_ref[...] + beta_ref[...]


def kernel(x, edge_index, edge_attr, u, batch, W1, b1, W2, b2, gamma, beta):
    edge_src = edge_index[0]
    esum, nsum, ecnt, ncnt = _sc_segment_sums(edge_src, edge_attr.T, x, batch)
    w1u = W1[:D_GLOB]
    w1e = W1[D_GLOB:D_GLOB + D_EDGE]
    w1n = W1[D_GLOB + D_EDGE:]
    return pl.pallas_call(
        _tc_body,
        out_shape=jax.ShapeDtypeStruct((B, D_GLOB), jnp.float32),
    )(u, esum, nsum, ecnt, ncnt, w1u, w1e, w1n,
      b1[None, :], W2, b2[None, :], gamma[None, :], beta[None, :])


# speedup vs baseline: 3.1428x; 3.1428x over previous
import functools

import jax
import jax.numpy as jnp
from jax import lax
from jax.experimental import pallas as pl
from jax.experimental.pallas import tpu as pltpu
from jax.experimental.pallas import tpu_sc as plsc

N = 100000
E = 1600000
D_NODE = 128
D_EDGE = 16
D_GLOB = 64
B = 256

NC = 2
NS = 16
NW = NC * NS
L = 16

UE = 256
EU = E // UE
K_E = (EU + NW - 1) // NW
UN = 200
NU = N // UN
K_N = (NU + NW - 1) // NW


def _sc_body(src_hbm, attr_hbm, x_hbm, batch_hbm, ze_hbm, zn_hbm, zf_hbm,
             esum_hbm, nsum_hbm, ecnt_hbm, ncnt_hbm,
             batch_spm, eacc_spm, nacc_spm,
             idx0, idx1, seg0, seg1, at0, at1, ar0, ar1, x0, nb0,
             hist, cnt_v, si0, si1, sa0, sa1, sg0, sg1, ss0, ss1):
    c = lax.axis_index("c")
    s = lax.axis_index("s")
    wid = s * NC + c

    @pl.when(s == 0)
    def _():
        pltpu.sync_copy(batch_hbm, batch_spm)

    pltpu.sync_copy(ze_hbm, eacc_spm.at[pl.ds(s * (B // NS), B // NS)])
    pltpu.sync_copy(zn_hbm, nacc_spm.at[pl.ds(s * (B // NS), B // NS)])
    pltpu.sync_copy(zf_hbm, hist)
    plsc.subcore_barrier()

    lane = lax.broadcasted_iota(jnp.int32, (L,), 0) * B
    rows16 = lax.broadcasted_iota(jnp.int32, (L,), 0)
    ones = jnp.ones((L,), jnp.int32)

    def histo(segb, n):
        for i in range(n // L):
            segs = segb[pl.ds(i * L, L)]
            plsc.addupdate_scatter(hist, [lane + segs], ones)

    def cnt_out(out):
        for b in range(B // L):
            acc = hist[pl.ds(b * L, L)]
            for l in range(1, L):
                acc = acc + hist[pl.ds(l * B + b * L, L)]
            cnt_v[pl.ds(b * L, L)] = acc
        pltpu.sync_copy(cnt_v, out.at[wid])

    def transpose(atb, arb):
        def gbody(g, carry):
            off = pl.multiple_of(g * L, L)
            rows = rows16 + g * L
            for f in range(D_EDGE):
                vals = atb[f, pl.ds(off, L)]
                plsc.store_scatter(arb, [rows, jnp.full((L,), f, jnp.int32)],
                                   vals)
            return carry

        lax.fori_loop(0, UE // L, gbody, 0)

    def e_loads(t, idxb, atb, s_i, s_a):
        base = pl.multiple_of(t * UE, UE)
        di = pltpu.async_copy(src_hbm.at[pl.ds(base, UE)], idxb, s_i)
        da = pltpu.async_copy(attr_hbm.at[:, pl.ds(base, UE)], atb, s_a)
        return di, da

    def edge_pair(j, carry):
        t0 = (2 * j) * NW + wid
        t1 = t0 + NW
        di0, da0 = e_loads(t0, idx0, at0, si0, sa0)
        di0.wait()
        dg0 = pltpu.async_copy(batch_spm.at[idx0], seg0, sg0)

        @pl.when(t1 < EU)
        def _():
            e_loads(t1, idx1, at1, si1, sa1)

        da0.wait()
        transpose(at0, ar0)
        dg0.wait()
        ds0 = pltpu.async_copy(ar0, eacc_spm.at[seg0], ss0, add=True)
        histo(seg0, UE)

        @pl.when(t1 < EU)
        def _():
            base = pl.multiple_of(t1 * UE, UE)
            pltpu.make_async_copy(src_hbm.at[pl.ds(base, UE)], idx1, si1).wait()
            dg1 = pltpu.async_copy(batch_spm.at[idx1], seg1, sg1)
            pltpu.make_async_copy(attr_hbm.at[:, pl.ds(base, UE)], at1,
                                  sa1).wait()
            transpose(at1, ar1)
            dg1.wait()
            ds1 = pltpu.async_copy(ar1, eacc_spm.at[seg1], ss1, add=True)
            histo(seg1, UE)
            ds1.wait()

        ds0.wait()
        return carry

    lax.fori_loop(0, K_E // 2, edge_pair, 0)

    cnt_out(ecnt_hbm)
    pltpu.sync_copy(zf_hbm, hist)

    def n_loads(t, nbb, xb, s_i, s_a):
        base = pl.multiple_of(t * UN, UN)
        dn = pltpu.async_copy(batch_hbm.at[pl.ds(base, UN)], nbb, s_i)
        dx = pltpu.async_copy(x_hbm.at[pl.ds(base, UN)], xb, s_a)
        return dn, dx

    def node_step(k, carry):
        t = k * NW + wid

        @pl.when(t < NU)
        def _():
            dn0, dx0 = n_loads(t, nb0, x0, si0, sa0)
            dn0.wait()
            dx0.wait()
            ds0 = pltpu.async_copy(x0, nacc_spm.at[nb0], ss0, add=True)
            histo(nb0, UN)
            ds0.wait()

        return carry

    lax.fori_loop(0, K_N, node_step, 0)

    cnt_out(ncnt_hbm)

    plsc.subcore_barrier()

    @pl.when(s == 0)
    def _():
        pltpu.sync_copy(eacc_spm, esum_hbm.at[c])
        pltpu.sync_copy(nacc_spm, nsum_hbm.at[c])


def _sc_segment_sums(edge_src, edge_attr_t, x, batch):
    mesh = plsc.VectorSubcoreMesh(core_axis_name="c", subcore_axis_name="s",
                                  num_cores=NC, num_subcores=NS)
    ze = jnp.zeros((B // NS, D_EDGE), jnp.float32)
    zn = jnp.zeros((B // NS, D_NODE), jnp.float32)
    zf = jnp.zeros((L * B,), jnp.int32)
    out_type = (
        jax.ShapeDtypeStruct((NC, B, D_EDGE), jnp.float32),
        jax.ShapeDtypeStruct((NC, B, D_NODE), jnp.float32),
        jax.ShapeDtypeStruct((NW, B), jnp.int32),
        jax.ShapeDtypeStruct((NW, B), jnp.int32),
    )
    scratch = [
        pltpu.VMEM_SHARED((N,), jnp.int32),
        pltpu.VMEM_SHARED((B, D_EDGE), jnp.float32),
        pltpu.VMEM_SHARED((B, D_NODE), jnp.float32),
        pltpu.VMEM((UE,), jnp.int32),
        pltpu.VMEM((UE,), jnp.int32),
        pltpu.VMEM((UE,), jnp.int32),
        pltpu.VMEM((UE,), jnp.int32),
        pltpu.VMEM((D_EDGE, UE), jnp.float32),
        pltpu.VMEM((D_EDGE, UE), jnp.float32),
        pltpu.VMEM((UE, D_EDGE), jnp.float32),
        pltpu.VMEM((UE, D_EDGE), jnp.float32),
        pltpu.VMEM((UN, D_NODE), jnp.float32),
        pltpu.VMEM((UN,), jnp.int32),
        pltpu.VMEM((L * B,), jnp.int32),
        pltpu.VMEM((B,), jnp.int32),
        pltpu.SemaphoreType.DMA,
        pltpu.SemaphoreType.DMA,
        pltpu.SemaphoreType.DMA,
        pltpu.SemaphoreType.DMA,
        pltpu.SemaphoreType.DMA,
        pltpu.SemaphoreType.DMA,
        pltpu.SemaphoreType.DMA,
        pltpu.SemaphoreType.DMA,
    ]
    fn = pl.kernel(_sc_body, out_type=out_type, mesh=mesh,
                   scratch_types=scratch,
                   compiler_params=pltpu.CompilerParams(
                       needs_layout_passes=False))
    return fn(edge_src, edge_attr_t, x, batch, ze, zn, zf)


def _tc_body(u_ref, esum_ref, nsum_ref, ecnt_ref, ncnt_ref,
             w1u_ref, w1e_ref, w1n_ref, b1_ref, w2_ref, b2_ref,
             gamma_ref, beta_ref, out_ref):
    e_sum = esum_ref[0] + esum_ref[1]
    n_sum = nsum_ref[0] + nsum_ref[1]
    e_cnt = jnp.sum(ecnt_ref[...], axis=0).astype(jnp.float32)
    n_cnt = jnp.sum(ncnt_ref[...], axis=0).astype(jnp.float32)
    e_mean = e_sum / jnp.maximum(e_cnt, 1.0)[:, None]
    n_mean = n_sum / jnp.maximum(n_cnt, 1.0)[:, None]
    u = u_ref[...]
    h = (jnp.dot(u, w1u_ref[...], preferred_element_type=jnp.float32)
         + jnp.dot(e_mean, w1e_ref[...], preferred_element_type=jnp.float32)
         + jnp.dot(n_mean, w1n_ref[...], preferred_element_type=jnp.float32)
         + b1_ref[...])
    h = jnp.maximum(h, 0.0)
    y = jnp.dot(h, w2_ref[...], preferred_element_type=jnp.float32) + b2_ref[...] + u
    mu = jnp.mean(y, axis=-1, keepdims=True)
    var = jnp.mean((y - mu) ** 2, axis=-1, keepdims=True)
    y = (y - mu) * lax.rsqrt(var + 1e-5)
    out_ref[...] = y * gamma_ref[...] + beta_ref[...]


def kernel(x, edge_index, edge_attr, u, batch, W1, b1, W2, b2, gamma, beta):
    edge_src = edge_index[0]
    esum, nsum, ecnt, ncnt = _sc_segment_sums(edge_src, edge_attr.T, x, batch)
    w1u = W1[:D_GLOB]
    w1e = W1[D_GLOB:D_GLOB + D_EDGE]
    w1n = W1[D_GLOB + D_EDGE:]
    return pl.pallas_call(
        _tc_body,
        out_shape=jax.ShapeDtypeStruct((B, D_GLOB), jnp.float32),
    )(u, esum, nsum, ecnt, ncnt, w1u, w1e, w1n,
      b1[None, :], W2, b2[None, :], gamma[None, :], beta[None, :])


# speedup vs baseline: 3.5955x; 1.1441x over previous
import functools

import jax
import jax.numpy as jnp
from jax import lax
from jax.experimental import pallas as pl
from jax.experimental.pallas import tpu as pltpu
from jax.experimental.pallas import tpu_sc as plsc

N = 100000
E = 1600000
D_NODE = 128
D_EDGE = 16
D_GLOB = 64
B = 256

NC = 2
NS = 16
NW = NC * NS
L = 16

UE = 256
EU = E // UE
K_E = (EU + NW - 1) // NW
UN = 200
NU = N // UN
K_N = (NU + NW - 1) // NW


def _sc_body(src_hbm, attr_hbm, x_hbm, batch_hbm, ze_hbm, zn_hbm, zf_hbm,
             esum_hbm, nsum_hbm, ecnt_hbm, ncnt_hbm,
             batch_spm, eacc_spm, nacc_spm,
             idx0, idx1, seg0, seg1, at0, at1, ar0, ar1, x0, nb0,
             hist, cnt_v, si0, si1, sa0, sa1, sg0, sg1, ss0, ss1):
    c = lax.axis_index("c")
    s = lax.axis_index("s")
    wid = s * NC + c

    @pl.when(s == 0)
    def _():
        pltpu.sync_copy(batch_hbm, batch_spm)

    pltpu.sync_copy(ze_hbm, eacc_spm.at[pl.ds(s * (B // NS), B // NS)])
    pltpu.sync_copy(zn_hbm, nacc_spm.at[pl.ds(s * (B // NS), B // NS)])
    pltpu.sync_copy(zf_hbm, hist)
    plsc.subcore_barrier()

    lane = lax.broadcasted_iota(jnp.int32, (L,), 0) * B
    rows16 = lax.broadcasted_iota(jnp.int32, (L,), 0)
    ones = jnp.ones((L,), jnp.int32)

    def histo(segb, n):
        for i in range(n // L):
            segs = segb[pl.ds(i * L, L)]
            plsc.addupdate_scatter(hist, [lane + segs], ones)

    def cnt_out(out):
        for b in range(B // L):
            acc = hist[pl.ds(b * L, L)]
            for l in range(1, L):
                acc = acc + hist[pl.ds(l * B + b * L, L)]
            cnt_v[pl.ds(b * L, L)] = acc
        pltpu.sync_copy(cnt_v, out.at[wid])

    def transpose(atb, arb):
        @plsc.parallel_loop(0, UE // L, unroll=4)
        def _(g):
            off = pl.multiple_of(g * L, L)
            rows = rows16 + g * L
            for f in range(D_EDGE):
                vals = atb[f, pl.ds(off, L)]
                plsc.store_scatter(arb, [rows, jnp.full((L,), f, jnp.int32)],
                                   vals)

    def e_loads(t, idxb, atb, s_i, s_a):
        base = pl.multiple_of(t * UE, UE)
        di = pltpu.async_copy(src_hbm.at[pl.ds(base, UE)], idxb, s_i)
        da = pltpu.async_copy(attr_hbm.at[:, pl.ds(base, UE)], atb, s_a)
        return di, da

    def edge_pair(j, carry):
        t0 = (2 * j) * NW + wid
        t1 = t0 + NW
        di0, da0 = e_loads(t0, idx0, at0, si0, sa0)
        di0.wait()
        dg0 = pltpu.async_copy(batch_spm.at[idx0], seg0, sg0)

        @pl.when(t1 < EU)
        def _():
            e_loads(t1, idx1, at1, si1, sa1)

        da0.wait()
        transpose(at0, ar0)
        dg0.wait()
        ds0 = pltpu.async_copy(ar0, eacc_spm.at[seg0], ss0, add=True)
        histo(seg0, UE)

        @pl.when(t1 < EU)
        def _():
            base = pl.multiple_of(t1 * UE, UE)
            pltpu.make_async_copy(src_hbm.at[pl.ds(base, UE)], idx1, si1).wait()
            dg1 = pltpu.async_copy(batch_spm.at[idx1], seg1, sg1)
            pltpu.make_async_copy(attr_hbm.at[:, pl.ds(base, UE)], at1,
                                  sa1).wait()
            transpose(at1, ar1)
            dg1.wait()
            ds1 = pltpu.async_copy(ar1, eacc_spm.at[seg1], ss1, add=True)
            histo(seg1, UE)
            ds1.wait()

        ds0.wait()
        return carry

    lax.fori_loop(0, K_E // 2, edge_pair, 0)

    cnt_out(ecnt_hbm)
    pltpu.sync_copy(zf_hbm, hist)

    def n_loads(t, nbb, xb, s_i, s_a):
        base = pl.multiple_of(t * UN, UN)
        dn = pltpu.async_copy(batch_hbm.at[pl.ds(base, UN)], nbb, s_i)
        dx = pltpu.async_copy(x_hbm.at[pl.ds(base, UN)], xb, s_a)
        return dn, dx

    def node_step(k, carry):
        t = k * NW + wid

        @pl.when(t < NU)
        def _():
            dn0, dx0 = n_loads(t, nb0, x0, si0, sa0)
            dn0.wait()
            dx0.wait()
            ds0 = pltpu.async_copy(x0, nacc_spm.at[nb0], ss0, add=True)
            histo(nb0, UN)
            ds0.wait()

        return carry

    lax.fori_loop(0, K_N, node_step, 0)

    cnt_out(ncnt_hbm)

    plsc.subcore_barrier()

    @pl.when(s == 0)
    def _():
        pltpu.sync_copy(eacc_spm, esum_hbm.at[c])
        pltpu.sync_copy(nacc_spm, nsum_hbm.at[c])


def _sc_segment_sums(edge_src, edge_attr_t, x, batch):
    mesh = plsc.VectorSubcoreMesh(core_axis_name="c", subcore_axis_name="s",
                                  num_cores=NC, num_subcores=NS)
    ze = jnp.zeros((B // NS, D_EDGE), jnp.float32)
    zn = jnp.zeros((B // NS, D_NODE), jnp.float32)
    zf = jnp.zeros((L * B,), jnp.int32)
    out_type = (
        jax.ShapeDtypeStruct((NC, B, D_EDGE), jnp.float32),
        jax.ShapeDtypeStruct((NC, B, D_NODE), jnp.float32),
        jax.ShapeDtypeStruct((NW, B), jnp.int32),
        jax.ShapeDtypeStruct((NW, B), jnp.int32),
    )
    scratch = [
        pltpu.VMEM_SHARED((N,), jnp.int32),
        pltpu.VMEM_SHARED((B, D_EDGE), jnp.float32),
        pltpu.VMEM_SHARED((B, D_NODE), jnp.float32),
        pltpu.VMEM((UE,), jnp.int32),
        pltpu.VMEM((UE,), jnp.int32),
        pltpu.VMEM((UE,), jnp.int32),
        pltpu.VMEM((UE,), jnp.int32),
        pltpu.VMEM((D_EDGE, UE), jnp.float32),
        pltpu.VMEM((D_EDGE, UE), jnp.float32),
        pltpu.VMEM((UE, D_EDGE), jnp.float32),
        pltpu.VMEM((UE, D_EDGE), jnp.float32),
        pltpu.VMEM((UN, D_NODE), jnp.float32),
        pltpu.VMEM((UN,), jnp.int32),
        pltpu.VMEM((L * B,), jnp.int32),
        pltpu.VMEM((B,), jnp.int32),
        pltpu.SemaphoreType.DMA,
        pltpu.SemaphoreType.DMA,
        pltpu.SemaphoreType.DMA,
        pltpu.SemaphoreType.DMA,
        pltpu.SemaphoreType.DMA,
        pltpu.SemaphoreType.DMA,
        pltpu.SemaphoreType.DMA,
        pltpu.SemaphoreType.DMA,
    ]
    fn = pl.kernel(_sc_body, out_type=out_type, mesh=mesh,
                   scratch_types=scratch,
                   compiler_params=pltpu.CompilerParams(
                       needs_layout_passes=False))
    return fn(edge_src, edge_attr_t, x, batch, ze, zn, zf)


def _tc_body(u_ref, esum_ref, nsum_ref, ecnt_ref, ncnt_ref,
             w1u_ref, w1e_ref, w1n_ref, b1_ref, w2_ref, b2_ref,
             gamma_ref, beta_ref, out_ref):
    e_sum = esum_ref[0] + esum_ref[1]
    n_sum = nsum_ref[0] + nsum_ref[1]
    e_cnt = jnp.sum(ecnt_ref[...], axis=0).astype(jnp.float32)
    n_cnt = jnp.sum(ncnt_ref[...], axis=0).astype(jnp.float32)
    e_mean = e_sum / jnp.maximum(e_cnt, 1.0)[:, None]
    n_mean = n_sum / jnp.maximum(n_cnt, 1.0)[:, None]
    u = u_ref[...]
    h = (jnp.dot(u, w1u_ref[...], preferred_element_type=jnp.float32)
         + jnp.dot(e_mean, w1e_ref[...], preferred_element_type=jnp.float32)
         + jnp.dot(n_mean, w1n_ref[...], preferred_element_type=jnp.float32)
         + b1_ref[...])
    h = jnp.maximum(h, 0.0)
    y = jnp.dot(h, w2_ref[...], preferred_element_type=jnp.float32) + b2_ref[...] + u
    mu = jnp.mean(y, axis=-1, keepdims=True)
    var = jnp.mean((y - mu) ** 2, axis=-1, keepdims=True)
    y = (y - mu) * lax.rsqrt(var + 1e-5)
    out_ref[...] = y * gamma_ref[...] + beta_ref[...]


def kernel(x, edge_index, edge_attr, u, batch, W1, b1, W2, b2, gamma, beta):
    edge_src = edge_index[0]
    esum, nsum, ecnt, ncnt = _sc_segment_sums(edge_src, edge_attr.T, x, batch)
    w1u = W1[:D_GLOB]
    w1e = W1[D_GLOB:D_GLOB + D_EDGE]
    w1n = W1[D_GLOB + D_EDGE:]
    return pl.pallas_call(
        _tc_body,
        out_shape=jax.ShapeDtypeStruct((B, D_GLOB), jnp.float32),
    )(u, esum, nsum, ecnt, ncnt, w1u, w1e, w1n,
      b1[None, :], W2, b2[None, :], gamma[None, :], beta[None, :])
